# baseline (speedup 1.0000x reference)
import functools

import jax
import jax.numpy as jnp
from jax import lax
from jax.experimental import pallas as pl
from jax.experimental.pallas import tpu as pltpu
from jax.experimental.pallas import tpu_sc as plsc

NUM_EMBEDDINGS = 1024
EMBEDDING_DIM = 64
COMMITMENT_COST = 0.25

N_ROWS = 32 * 576
N_HALF = N_ROWS // 2
BLK = 1024
NBLK = N_HALF // BLK

NW = 32
BPW = N_HALF // NW
CHUNK = 96
NCHUNK = BPW // CHUNK


def _vq_block(x, w, ii):
    ones64 = jnp.ones((EMBEDDING_DIM, 1), jnp.float32)
    x2 = lax.dot_general(x * x, ones64, (((1,), (0,)), ((), ())))
    w2 = lax.dot_general(w * w, ones64, (((1,), (0,)), ((), ())))[:, 0]
    mm = lax.dot_general(x, w, (((1,), (1,)), ((), ())))
    d = x2 + w2 - 2.0 * mm
    m = jnp.min(d, axis=1, keepdims=True)
    idxf = jnp.min(jnp.where(d == m, ii, 1024.0), axis=1, keepdims=True)
    onehot = (idxf == ii).astype(jnp.float32)
    ones_row = jnp.ones((1, BLK), jnp.float32)
    hist = lax.dot_general(ones_row, onehot, (((1,), (0,)), ((), ())))
    return idxf[:, 0].astype(jnp.int32), jnp.sum(m), hist


def _tc_body_a(x_ref, w_ref, iota_ref, idx_ref, lsum_o, cnt_o, cnt_s, lsum_s):
    b = pl.program_id(0)

    @pl.when(b == 0)
    def _init():
        cnt_s[...] = jnp.zeros_like(cnt_s)
        lsum_s[0, 0] = 0.0

    idx, msum, hist = _vq_block(x_ref[...], w_ref[...], iota_ref[...])
    idx_ref[0, 0, :] = idx
    cnt_s[...] += hist
    lsum_s[0, 0] += msum

    @pl.when(b == NBLK - 1)
    def _fini():
        lsum_o[0, 0] = lsum_s[0, 0]
        cnt_o[...] = cnt_s[...]


def _tc_body_b(x_ref, w_ref, iota_ref, lin_ref, cin_ref, idx_ref, loss_ref,
               perp_ref, cnt_s, lsum_s):
    b = pl.program_id(0)

    @pl.when(b == 0)
    def _init():
        cnt_s[...] = cin_ref[...]
        lsum_s[0, 0] = lin_ref[0, 0]

    idx, msum, hist = _vq_block(x_ref[...], w_ref[...], iota_ref[...])
    idx_ref[0, 0, :] = idx
    cnt_s[...] += hist
    lsum_s[0, 0] += msum

    @pl.when(b == NBLK - 1)
    def _fini():
        mse = lsum_s[0, 0] / float(N_ROWS * EMBEDDING_DIM)
        loss_ref[0, 0] = mse + COMMITMENT_COST * mse
        p = cnt_s[...] / float(N_ROWS)
        ent = jnp.sum(p * jnp.log(p + 1e-10))
        perp_ref[0, 0] = jnp.exp(-ent)


_XSPEC = [
    pl.BlockSpec((BLK, EMBEDDING_DIM), lambda i: (i, 0)),
    pl.BlockSpec((NUM_EMBEDDINGS, EMBEDDING_DIM), lambda i: (0, 0)),
    pl.BlockSpec((1, NUM_EMBEDDINGS), lambda i: (0, 0)),
]
_IDXSPEC = pl.BlockSpec((1, 1, BLK), lambda i: (i, 0, 0))
_SCALSPEC = pl.BlockSpec((1, 1), lambda i: (0, 0), memory_space=pltpu.SMEM)
_CNTSPEC = pl.BlockSpec((1, NUM_EMBEDDINGS), lambda i: (0, 0))
_SCRATCH = [
    pltpu.VMEM((1, NUM_EMBEDDINGS), jnp.float32),
    pltpu.SMEM((1, 1), jnp.float32),
]
_PARAMS = pltpu.CompilerParams(dimension_semantics=("arbitrary",))


def _vq_tc_a(xh, W, iota):
    return pl.pallas_call(
        _tc_body_a,
        grid=(NBLK,),
        in_specs=_XSPEC,
        out_specs=[_IDXSPEC, _SCALSPEC, _CNTSPEC],
        out_shape=[
            jax.ShapeDtypeStruct((NBLK, 1, BLK), jnp.int32),
            jax.ShapeDtypeStruct((1, 1), jnp.float32),
            jax.ShapeDtypeStruct((1, NUM_EMBEDDINGS), jnp.float32),
        ],
        scratch_shapes=_SCRATCH,
        compiler_params=_PARAMS,
    )(xh, W, iota)


def _vq_tc_b(xh, W, iota, lsum_a, cnt_a):
    return pl.pallas_call(
        _tc_body_b,
        grid=(NBLK,),
        in_specs=_XSPEC + [_SCALSPEC, _CNTSPEC],
        out_specs=[_IDXSPEC, _SCALSPEC, _SCALSPEC],
        out_shape=[
            jax.ShapeDtypeStruct((NBLK, 1, BLK), jnp.int32),
            jax.ShapeDtypeStruct((1, 1), jnp.float32),
            jax.ShapeDtypeStruct((1, 1), jnp.float32),
        ],
        scratch_shapes=_SCRATCH,
        compiler_params=_PARAMS,
    )(xh, W, iota, lsum_a, cnt_a)


@functools.cache
def _make_sc_gather():
    mesh = plsc.VectorSubcoreMesh(core_axis_name="c", subcore_axis_name="s")

    @functools.partial(
        pl.kernel,
        mesh=mesh,
        out_type=jax.ShapeDtypeStruct((N_HALF, 128), jnp.float32),
        scratch_types=[
            pltpu.VMEM((BPW,), jnp.int32),
            pltpu.VMEM((BPW, 128), jnp.float32),
            pltpu.SemaphoreType.DMA,
        ],
    )
    def _sc_gather(table_hbm, idx_hbm, out_hbm, idx_v, rows_v, sem):
        wid = lax.axis_index("s") * 2 + lax.axis_index("c")
        base = wid * BPW
        pltpu.sync_copy(idx_hbm.at[pl.ds(base, BPW)], idx_v)
        copies = [
            pltpu.async_copy(
                table_hbm.at[idx_v.at[pl.ds(c * CHUNK, CHUNK)]],
                rows_v.at[pl.ds(c * CHUNK, CHUNK)],
                sem,
            )
            for c in range(NCHUNK)
        ]
        for cp in copies:
            cp.wait()
        pltpu.sync_copy(rows_v, out_hbm.at[pl.ds(base, BPW)])

    return _sc_gather


def kernel(inputs, W):
    input_shape = inputs.shape
    x = inputs.reshape(-1, EMBEDDING_DIM)
    iota = jnp.arange(NUM_EMBEDDINGS, dtype=jnp.float32).reshape(1, -1)
    table128 = jnp.concatenate(
        [W, jnp.zeros((NUM_EMBEDDINGS, 128 - EMBEDDING_DIM), jnp.float32)],
        axis=1)
    gather = _make_sc_gather()

    idx_a, lsum_a, cnt_a = _vq_tc_a(x[:N_HALF], W, iota)
    q_a = gather(table128, idx_a.reshape(-1))
    idx_b, loss11, perp11 = _vq_tc_b(x[N_HALF:], W, iota, lsum_a, cnt_a)
    q_b = gather(table128, idx_b.reshape(-1))

    quantized = jnp.concatenate(
        [q_a[:, :EMBEDDING_DIM], q_b[:, :EMBEDDING_DIM]], axis=0)
    indices = jnp.concatenate(
        [idx_a.reshape(-1), idx_b.reshape(-1)], axis=0)
    return (
        loss11.reshape(()),
        quantized.reshape(input_shape),
        perp11.reshape(()),
        indices.reshape(input_shape[0], -1),
    )


# speedup vs baseline: 1.2902x; 1.2902x over previous
import functools

import jax
import jax.numpy as jnp
from jax import lax
from jax.experimental import pallas as pl
from jax.experimental.pallas import tpu as pltpu
from jax.experimental.pallas import tpu_sc as plsc

NUM_EMBEDDINGS = 1024
EMBEDDING_DIM = 64
COMMITMENT_COST = 0.25

N_ROWS = 32 * 576
BLK = 1024
NBLK = N_ROWS // BLK

NW = 32
BPW = N_ROWS // NW
CHUNK = 96
NCHUNK = BPW // CHUNK


def _tc_body(x_ref, w_ref, iota_ref, idx_ref, loss_ref, perp_ref, cnt_s,
             lsum_s):
    b = pl.program_id(0)

    @pl.when(b == 0)
    def _init():
        cnt_s[...] = jnp.zeros_like(cnt_s)
        lsum_s[0, 0] = 0.0

    x = x_ref[...]
    w = w_ref[...]
    ii = iota_ref[...]
    ones64 = jnp.ones((EMBEDDING_DIM, 1), jnp.float32)
    x2 = lax.dot_general(x * x, ones64, (((1,), (0,)), ((), ())))
    w2 = lax.dot_general(w * w, ones64, (((1,), (0,)), ((), ())))[:, 0]
    mm = lax.dot_general(x, w, (((1,), (1,)), ((), ())))
    d = x2 + w2 - 2.0 * mm
    m = jnp.min(d, axis=1, keepdims=True)
    lsum_s[0, 0] += jnp.sum(m)
    idxf = jnp.min(jnp.where(d == m, ii, 1024.0), axis=1, keepdims=True)
    idx_ref[0, 0, :] = idxf[:, 0].astype(jnp.int32)
    onehot = (idxf == ii).astype(jnp.float32)
    ones_row = jnp.ones((1, BLK), jnp.float32)
    cnt_s[...] += lax.dot_general(ones_row, onehot, (((1,), (0,)), ((), ())))

    @pl.when(b == NBLK - 1)
    def _fini():
        mse = lsum_s[0, 0] / float(N_ROWS * EMBEDDING_DIM)
        loss_ref[0, 0] = mse + COMMITMENT_COST * mse
        p = cnt_s[...] / float(N_ROWS)
        ent = jnp.sum(p * jnp.log(p + 1e-10))
        perp_ref[0, 0] = jnp.exp(-ent)


def _vq_tc(x, W):
    return pl.pallas_call(
        _tc_body,
        grid=(NBLK,),
        in_specs=[
            pl.BlockSpec((BLK, EMBEDDING_DIM), lambda i: (i, 0)),
            pl.BlockSpec((NUM_EMBEDDINGS, EMBEDDING_DIM), lambda i: (0, 0)),
            pl.BlockSpec((1, NUM_EMBEDDINGS), lambda i: (0, 0)),
        ],
        out_specs=[
            pl.BlockSpec((1, 1, BLK), lambda i: (i, 0, 0)),
            pl.BlockSpec((1, 1), lambda i: (0, 0), memory_space=pltpu.SMEM),
            pl.BlockSpec((1, 1), lambda i: (0, 0), memory_space=pltpu.SMEM),
        ],
        out_shape=[
            jax.ShapeDtypeStruct((NBLK, 1, BLK), jnp.int32),
            jax.ShapeDtypeStruct((1, 1), jnp.float32),
            jax.ShapeDtypeStruct((1, 1), jnp.float32),
        ],
        scratch_shapes=[
            pltpu.VMEM((1, NUM_EMBEDDINGS), jnp.float32),
            pltpu.SMEM((1, 1), jnp.float32),
        ],
        compiler_params=pltpu.CompilerParams(
            dimension_semantics=("arbitrary",)),
    )(x, W, jnp.arange(NUM_EMBEDDINGS, dtype=jnp.float32).reshape(1, -1))


@functools.cache
def _make_sc_gather():
    mesh = plsc.VectorSubcoreMesh(core_axis_name="c", subcore_axis_name="s")

    @functools.partial(
        pl.kernel,
        mesh=mesh,
        out_type=jax.ShapeDtypeStruct((N_ROWS, 128), jnp.float32),
        scratch_types=[
            pltpu.VMEM((BPW,), jnp.int32),
            pltpu.VMEM((BPW, 128), jnp.float32),
            pltpu.SemaphoreType.DMA,
        ],
    )
    def _sc_gather(table_hbm, idx_hbm, out_hbm, idx_v, rows_v, sem):
        wid = lax.axis_index("s") * 2 + lax.axis_index("c")
        base = wid * BPW
        pltpu.sync_copy(idx_hbm.at[pl.ds(base, BPW)], idx_v)
        copies = [
            pltpu.async_copy(
                table_hbm.at[idx_v.at[pl.ds(c * CHUNK, CHUNK)]],
                rows_v.at[pl.ds(c * CHUNK, CHUNK)],
                sem,
            )
            for c in range(NCHUNK)
        ]
        for cp in copies:
            cp.wait()
        pltpu.sync_copy(rows_v, out_hbm.at[pl.ds(base, BPW)])

    return _sc_gather


def kernel(inputs, W):
    input_shape = inputs.shape
    x = inputs.reshape(-1, EMBEDDING_DIM)
    idx3, loss11, perp11 = _vq_tc(x, W)
    idx_flat = idx3.reshape(-1)
    table128 = jnp.concatenate(
        [W, jnp.zeros((NUM_EMBEDDINGS, 128 - EMBEDDING_DIM), jnp.float32)],
        axis=1)
    quantized = _make_sc_gather()(table128, idx_flat)[:, :EMBEDDING_DIM]
    return (
        loss11.reshape(()),
        quantized.reshape(input_shape),
        perp11.reshape(()),
        idx3.reshape(input_shape[0], -1),
    )


# speedup vs baseline: 1.3144x; 1.0187x over previous
import functools

import jax
import jax.numpy as jnp
from jax import lax
from jax.experimental import pallas as pl
from jax.experimental.pallas import tpu as pltpu
from jax.experimental.pallas import tpu_sc as plsc

NUM_EMBEDDINGS = 1024
EMBEDDING_DIM = 64
COMMITMENT_COST = 0.25

N_ROWS = 32 * 576
BLK = 1024
NBLK = N_ROWS // BLK

NW = 32
BPW = N_ROWS // NW
CHUNK = 96
NCHUNK = BPW // CHUNK


def _tc_body(x_ref, w_ref, iota_ref, idx_ref, loss_ref, perp_ref, cnt_s,
             lsum_s, w2_s):
    b = pl.program_id(0)
    ones64 = jnp.ones((EMBEDDING_DIM, 1), jnp.float32)

    @pl.when(b == 0)
    def _init():
        cnt_s[...] = jnp.zeros_like(cnt_s)
        lsum_s[0, 0] = 0.0
        ww = w_ref[...]
        w2_s[...] = lax.dot_general(
            ww * ww, ones64, (((1,), (0,)), ((), ()))).reshape(1, -1)

    x = x_ref[...]
    w = w_ref[...]
    ii = iota_ref[...]
    x2 = lax.dot_general(x * x, ones64, (((1,), (0,)), ((), ())))
    w2 = w2_s[...][0]
    mm = lax.dot_general(x, w, (((1,), (1,)), ((), ())))
    d = x2 + w2 - 2.0 * mm
    m = jnp.min(d, axis=1, keepdims=True)
    lsum_s[0, 0] += jnp.sum(m)
    idxf = jnp.min(jnp.where(d == m, ii, 1024.0), axis=1, keepdims=True)
    idx_ref[0, 0, :] = idxf[:, 0].astype(jnp.int32)
    onehot = (idxf == ii).astype(jnp.float32)
    ones_row = jnp.ones((1, BLK), jnp.float32)
    cnt_s[...] += lax.dot_general(ones_row, onehot, (((1,), (0,)), ((), ())))

    @pl.when(b == NBLK - 1)
    def _fini():
        mse = lsum_s[0, 0] / float(N_ROWS * EMBEDDING_DIM)
        loss_ref[0, 0] = mse + COMMITMENT_COST * mse
        p = cnt_s[...] / float(N_ROWS)
        ent = jnp.sum(p * jnp.log(p + 1e-10))
        perp_ref[0, 0] = jnp.exp(-ent)


def _vq_tc(x, W):
    return pl.pallas_call(
        _tc_body,
        grid=(NBLK,),
        in_specs=[
            pl.BlockSpec((BLK, EMBEDDING_DIM), lambda i: (i, 0)),
            pl.BlockSpec((NUM_EMBEDDINGS, EMBEDDING_DIM), lambda i: (0, 0)),
            pl.BlockSpec((1, NUM_EMBEDDINGS), lambda i: (0, 0)),
        ],
        out_specs=[
            pl.BlockSpec((1, 1, BLK), lambda i: (i, 0, 0)),
            pl.BlockSpec((1, 1), lambda i: (0, 0), memory_space=pltpu.SMEM),
            pl.BlockSpec((1, 1), lambda i: (0, 0), memory_space=pltpu.SMEM),
        ],
        out_shape=[
            jax.ShapeDtypeStruct((NBLK, 1, BLK), jnp.int32),
            jax.ShapeDtypeStruct((1, 1), jnp.float32),
            jax.ShapeDtypeStruct((1, 1), jnp.float32),
        ],
        scratch_shapes=[
            pltpu.VMEM((1, NUM_EMBEDDINGS), jnp.float32),
            pltpu.SMEM((1, 1), jnp.float32),
            pltpu.VMEM((1, NUM_EMBEDDINGS), jnp.float32),
        ],
        compiler_params=pltpu.CompilerParams(
            dimension_semantics=("arbitrary",)),
    )(x, W, jnp.arange(NUM_EMBEDDINGS, dtype=jnp.float32).reshape(1, -1))


@functools.cache
def _make_sc_gather():
    mesh = plsc.VectorSubcoreMesh(core_axis_name="c", subcore_axis_name="s")

    @functools.partial(
        pl.kernel,
        mesh=mesh,
        out_type=jax.ShapeDtypeStruct((N_ROWS, 128), jnp.float32),
        scratch_types=[
            pltpu.VMEM((BPW,), jnp.int32),
            pltpu.VMEM((BPW, 128), jnp.float32),
            pltpu.SemaphoreType.DMA,
        ],
    )
    def _sc_gather(table_hbm, idx_hbm, out_hbm, idx_v, rows_v, sem):
        wid = lax.axis_index("s") * 2 + lax.axis_index("c")
        base = wid * BPW
        pltpu.sync_copy(idx_hbm.at[pl.ds(base, BPW)], idx_v)
        copies = [
            pltpu.async_copy(
                table_hbm.at[idx_v.at[pl.ds(c * CHUNK, CHUNK)]],
                rows_v.at[pl.ds(c * CHUNK, CHUNK)],
                sem,
            )
            for c in range(NCHUNK)
        ]
        for cp in copies:
            cp.wait()
        pltpu.sync_copy(rows_v, out_hbm.at[pl.ds(base, BPW)])

    return _sc_gather


def kernel(inputs, W):
    input_shape = inputs.shape
    x = inputs.reshape(-1, EMBEDDING_DIM)
    idx3, loss11, perp11 = _vq_tc(x, W)
    idx_flat = idx3.reshape(-1)
    table128 = jnp.concatenate(
        [W, jnp.zeros((NUM_EMBEDDINGS, 128 - EMBEDDING_DIM), jnp.float32)],
        axis=1)
    quantized = _make_sc_gather()(table128, idx_flat)[:, :EMBEDDING_DIM]
    return (
        loss11.reshape(()),
        quantized.reshape(input_shape),
        perp11.reshape(()),
        idx3.reshape(input_shape[0], -1),
    )
